# Initial kernel scaffold; baseline (speedup 1.0000x reference)
import jax
import jax.numpy as jnp
from jax.experimental import pallas as pl


def kernel(boxes, scores):
    raise NotImplementedError("write your pallas kernel here")


# speedup vs baseline: 192.8113x; 192.8113x over previous
import jax
import jax.numpy as jnp
from jax.experimental import pallas as pl
from jax.experimental.pallas import tpu as pltpu

N = 5000
NPAD = 5120
B = 256
NBLK = NPAD // B
THRESH = 0.3


def _nms_body(col_ref, row_ref, keep_ref):
    keep_ref[...] = jnp.ones((1, NPAD), jnp.float32)
    cx1 = row_ref[0:1, :]
    cy1 = row_ref[1:2, :]
    cx2 = row_ref[2:3, :]
    cy2 = row_ref[3:4, :]
    carea = row_ref[4:5, :]
    col_ids = jax.lax.broadcasted_iota(jnp.int32, (B, NPAD), 1)
    row_iota = jax.lax.broadcasted_iota(jnp.int32, (B, NPAD), 0)

    def block(b, carry):
        off = b * B
        rx1 = col_ref[pl.ds(off, B), 0:1]
        ry1 = col_ref[pl.ds(off, B), 1:2]
        rx2 = col_ref[pl.ds(off, B), 2:3]
        ry2 = col_ref[pl.ds(off, B), 3:4]
        rarea = col_ref[pl.ds(off, B), 4:5]
        iw = jnp.maximum(jnp.minimum(rx2, cx2) - jnp.maximum(rx1, cx1), 0.0)
        ih = jnp.maximum(jnp.minimum(ry2, cy2) - jnp.maximum(ry1, cy1), 0.0)
        inter = iw * ih
        union = rarea + carea - inter
        iou = inter / jnp.maximum(union, 1e-9)
        cand = (iou > THRESH) & (col_ids > row_iota + off)
        Sb = jnp.where(cand, 1.0, 0.0).astype(jnp.bfloat16)
        bx1 = row_ref[0:1, pl.ds(off, B)]
        by1 = row_ref[1:2, pl.ds(off, B)]
        bx2 = row_ref[2:3, pl.ds(off, B)]
        by2 = row_ref[3:4, pl.ds(off, B)]
        barea = row_ref[4:5, pl.ds(off, B)]
        biw = jnp.maximum(jnp.minimum(rx2, bx2) - jnp.maximum(rx1, bx1), 0.0)
        bih = jnp.maximum(jnp.minimum(ry2, by2) - jnp.maximum(ry1, by1), 0.0)
        binter = biw * bih
        bunion = rarea + barea - binter
        biou = binter / jnp.maximum(bunion, 1e-9)
        tri = (jax.lax.broadcasted_iota(jnp.int32, (B, B), 1)
               > jax.lax.broadcasted_iota(jnp.int32, (B, B), 0))
        S_bb = jnp.where((biou > THRESH) & tri, 1.0, 0.0).astype(jnp.bfloat16)
        k_in = keep_ref[0:1, pl.ds(off, B)]

        def cond(state):
            t, _, changed = state
            return changed & (t < B)

        def body(state):
            t, k, _ = state
            cnt = jax.lax.dot_general(
                k.astype(jnp.bfloat16), S_bb,
                (((1,), (0,)), ((), ())),
                preferred_element_type=jnp.float32)
            k_new = k_in * jnp.where(cnt > 0.0, 0.0, 1.0)
            return t + 1, k_new, jnp.any(k_new != k)

        _, k_fin, _ = jax.lax.while_loop(cond, body,
                                         (0, k_in, jnp.bool_(True)))
        sup = jax.lax.dot_general(
            k_fin.astype(jnp.bfloat16), Sb,
            (((1,), (0,)), ((), ())),
            preferred_element_type=jnp.float32)
        keep_ref[...] = keep_ref[...] * jnp.where(sup > 0.0, 0.0, 1.0)
        return carry

    jax.lax.fori_loop(0, NBLK, block, 0)


def _run_nms(colmat, rowmat, interpret=False):
    return pl.pallas_call(
        _nms_body,
        out_shape=jax.ShapeDtypeStruct((1, NPAD), jnp.float32),
        in_specs=[
            pl.BlockSpec(memory_space=pltpu.VMEM),
            pl.BlockSpec(memory_space=pltpu.VMEM),
        ],
        out_specs=pl.BlockSpec(memory_space=pltpu.VMEM),
        interpret=interpret,
    )(colmat, rowmat)


def kernel(boxes, scores, interpret=False):
    scores = scores.astype(jnp.float32)
    boxes = boxes.astype(jnp.float32)
    scores_p = jnp.concatenate(
        [scores, jnp.full((NPAD - N,), -1.0, jnp.float32)])
    boxes_p = jnp.concatenate(
        [boxes, jnp.zeros((NPAD - N, 4), jnp.float32)], axis=0)
    order = jnp.argsort(-scores_p)
    sb = boxes_p[order]
    x1, y1, x2, y2 = sb[:, 0], sb[:, 1], sb[:, 2], sb[:, 3]
    area = (x2 - x1) * (y2 - y1)
    zero = jnp.zeros((NPAD,), jnp.float32)
    colmat = jnp.stack([x1, y1, x2, y2, area, zero, zero, zero], axis=1)
    rowmat = colmat.T
    keep_sorted = _run_nms(colmat, rowmat, interpret=interpret)[0]
    keep = jnp.zeros((NPAD,), jnp.float32).at[order].set(keep_sorted)
    return scores * keep[:N]


# speedup vs baseline: 257.2460x; 1.3342x over previous
import jax
import jax.numpy as jnp
from jax.experimental import pallas as pl
from jax.experimental.pallas import tpu as pltpu

N = 5000
NPAD = 5120
B = 256
NBLK = NPAD // B
THRESH = 0.3


def _nms_body(col_ref, row_ref, keep_ref):
    keep_ref[...] = jnp.ones((1, NPAD), jnp.float32)
    tri = (jax.lax.broadcasted_iota(jnp.int32, (B, B), 1)
           > jax.lax.broadcasted_iota(jnp.int32, (B, B), 0))

    def block(b, carry):
        off = b * B
        rx1 = col_ref[pl.ds(off, B), 0:1]
        ry1 = col_ref[pl.ds(off, B), 1:2]
        rx2 = col_ref[pl.ds(off, B), 2:3]
        ry2 = col_ref[pl.ds(off, B), 3:4]
        rarea = col_ref[pl.ds(off, B), 4:5]

        def iou_chunk(coff):
            cx1 = row_ref[0:1, pl.ds(coff, B)]
            cy1 = row_ref[1:2, pl.ds(coff, B)]
            cx2 = row_ref[2:3, pl.ds(coff, B)]
            cy2 = row_ref[3:4, pl.ds(coff, B)]
            carea = row_ref[4:5, pl.ds(coff, B)]
            iw = jnp.maximum(jnp.minimum(rx2, cx2) - jnp.maximum(rx1, cx1),
                             0.0)
            ih = jnp.maximum(jnp.minimum(ry2, cy2) - jnp.maximum(ry1, cy1),
                             0.0)
            inter = iw * ih
            union = rarea + carea - inter
            return inter / jnp.maximum(union, 1e-9)

        S_bb = jnp.where((iou_chunk(off) > THRESH) & tri,
                         1.0, 0.0).astype(jnp.bfloat16)
        k_in = keep_ref[0:1, pl.ds(off, B)]

        def fcond(state):
            t, _, changed = state
            return changed & (t < B)

        def fbody(state):
            t, k, _ = state
            cnt = jax.lax.dot_general(
                k.astype(jnp.bfloat16), S_bb,
                (((1,), (0,)), ((), ())),
                preferred_element_type=jnp.float32)
            k_new = k_in * jnp.where(cnt > 0.0, 0.0, 1.0)
            return t + 1, k_new, jnp.any(k_new != k)

        _, k_fin, _ = jax.lax.while_loop(fcond, fbody,
                                         (0, k_in, jnp.bool_(True)))
        keep_ref[0:1, pl.ds(off, B)] = k_fin
        k_bf = k_fin.astype(jnp.bfloat16)

        def chunk(c, carry2):
            coff = c * B
            Sc = jnp.where(iou_chunk(coff) > THRESH,
                           1.0, 0.0).astype(jnp.bfloat16)
            sup = jax.lax.dot_general(
                k_bf, Sc, (((1,), (0,)), ((), ())),
                preferred_element_type=jnp.float32)
            keep_ref[0:1, pl.ds(coff, B)] = (
                keep_ref[0:1, pl.ds(coff, B)]
                * jnp.where(sup > 0.0, 0.0, 1.0))
            return carry2

        jax.lax.fori_loop(b + 1, NBLK, chunk, 0)
        return carry

    jax.lax.fori_loop(0, NBLK, block, 0)


def _run_nms(colmat, rowmat, interpret=False):
    return pl.pallas_call(
        _nms_body,
        out_shape=jax.ShapeDtypeStruct((1, NPAD), jnp.float32),
        in_specs=[
            pl.BlockSpec(memory_space=pltpu.VMEM),
            pl.BlockSpec(memory_space=pltpu.VMEM),
        ],
        out_specs=pl.BlockSpec(memory_space=pltpu.VMEM),
        interpret=interpret,
    )(colmat, rowmat)


def kernel(boxes, scores, interpret=False):
    scores = scores.astype(jnp.float32)
    boxes = boxes.astype(jnp.float32)
    scores_p = jnp.concatenate(
        [scores, jnp.full((NPAD - N,), -1.0, jnp.float32)])
    boxes_p = jnp.concatenate(
        [boxes, jnp.zeros((NPAD - N, 4), jnp.float32)], axis=0)
    order = jnp.argsort(-scores_p)
    sb = boxes_p[order]
    x1, y1, x2, y2 = sb[:, 0], sb[:, 1], sb[:, 2], sb[:, 3]
    area = (x2 - x1) * (y2 - y1)
    zero = jnp.zeros((NPAD,), jnp.float32)
    colmat = jnp.stack([x1, y1, x2, y2, area, zero, zero, zero], axis=1)
    rowmat = colmat.T
    keep_sorted = _run_nms(colmat, rowmat, interpret=interpret)[0]
    keep = jnp.zeros((NPAD,), jnp.float32).at[order].set(keep_sorted)
    return scores * keep[:N]


# speedup vs baseline: 313.7957x; 1.2198x over previous
import jax
import jax.numpy as jnp
from jax.experimental import pallas as pl
from jax.experimental.pallas import tpu as pltpu

N = 5000
NPAD = 5120
B = 512
NBLK = NPAD // B
THRESH = 0.3


def _nms_body(col_ref, row_ref, keep_ref):
    keep_ref[...] = jnp.ones((1, NPAD), jnp.float32)
    tri = (jax.lax.broadcasted_iota(jnp.int32, (B, B), 1)
           > jax.lax.broadcasted_iota(jnp.int32, (B, B), 0))

    def block(b, carry):
        off = b * B
        rx1 = col_ref[pl.ds(off, B), 0:1]
        ry1 = col_ref[pl.ds(off, B), 1:2]
        rx2 = col_ref[pl.ds(off, B), 2:3]
        ry2 = col_ref[pl.ds(off, B), 3:4]
        rarea = col_ref[pl.ds(off, B), 4:5]

        def iou_chunk(coff):
            cx1 = row_ref[0:1, pl.ds(coff, B)]
            cy1 = row_ref[1:2, pl.ds(coff, B)]
            cx2 = row_ref[2:3, pl.ds(coff, B)]
            cy2 = row_ref[3:4, pl.ds(coff, B)]
            carea = row_ref[4:5, pl.ds(coff, B)]
            iw = jnp.maximum(jnp.minimum(rx2, cx2) - jnp.maximum(rx1, cx1),
                             0.0)
            ih = jnp.maximum(jnp.minimum(ry2, cy2) - jnp.maximum(ry1, cy1),
                             0.0)
            inter = iw * ih
            union = rarea + carea - inter
            return inter / jnp.maximum(union, 1e-9)

        S_bb = jnp.where((iou_chunk(off) > THRESH) & tri,
                         1.0, 0.0).astype(jnp.bfloat16)
        k_in = keep_ref[0:1, pl.ds(off, B)]

        def fcond(state):
            t, _, changed = state
            return changed & (t < B)

        def fbody(state):
            t, k, _ = state
            cnt = jax.lax.dot_general(
                k.astype(jnp.bfloat16), S_bb,
                (((1,), (0,)), ((), ())),
                preferred_element_type=jnp.float32)
            k_new = k_in * jnp.where(cnt > 0.0, 0.0, 1.0)
            return t + 1, k_new, jnp.any(k_new != k)

        _, k_fin, _ = jax.lax.while_loop(fcond, fbody,
                                         (0, k_in, jnp.bool_(True)))
        keep_ref[0:1, pl.ds(off, B)] = k_fin
        k_bf = k_fin.astype(jnp.bfloat16)

        def chunk(c, carry2):
            coff = c * B
            Sc = jnp.where(iou_chunk(coff) > THRESH,
                           1.0, 0.0).astype(jnp.bfloat16)
            sup = jax.lax.dot_general(
                k_bf, Sc, (((1,), (0,)), ((), ())),
                preferred_element_type=jnp.float32)
            keep_ref[0:1, pl.ds(coff, B)] = (
                keep_ref[0:1, pl.ds(coff, B)]
                * jnp.where(sup > 0.0, 0.0, 1.0))
            return carry2

        jax.lax.fori_loop(b + 1, NBLK, chunk, 0)
        return carry

    jax.lax.fori_loop(0, NBLK, block, 0)


def _run_nms(colmat, rowmat, interpret=False):
    return pl.pallas_call(
        _nms_body,
        out_shape=jax.ShapeDtypeStruct((1, NPAD), jnp.float32),
        in_specs=[
            pl.BlockSpec(memory_space=pltpu.VMEM),
            pl.BlockSpec(memory_space=pltpu.VMEM),
        ],
        out_specs=pl.BlockSpec(memory_space=pltpu.VMEM),
        interpret=interpret,
    )(colmat, rowmat)


def kernel(boxes, scores, interpret=False):
    scores = scores.astype(jnp.float32)
    boxes = boxes.astype(jnp.float32)
    scores_p = jnp.concatenate(
        [scores, jnp.full((NPAD - N,), -1.0, jnp.float32)])
    boxes_p = jnp.concatenate(
        [boxes, jnp.zeros((NPAD - N, 4), jnp.float32)], axis=0)
    order = jnp.argsort(-scores_p)
    sb = boxes_p[order]
    x1, y1, x2, y2 = sb[:, 0], sb[:, 1], sb[:, 2], sb[:, 3]
    area = (x2 - x1) * (y2 - y1)
    zero = jnp.zeros((NPAD,), jnp.float32)
    colmat = jnp.stack([x1, y1, x2, y2, area, zero, zero, zero], axis=1)
    rowmat = colmat.T
    keep_sorted = _run_nms(colmat, rowmat, interpret=interpret)[0]
    keep = jnp.zeros((NPAD,), jnp.float32).at[order].set(keep_sorted)
    return scores * keep[:N]


# speedup vs baseline: 332.0004x; 1.0580x over previous
import jax
import jax.numpy as jnp
from jax.experimental import pallas as pl
from jax.experimental.pallas import tpu as pltpu

N = 5000
NPAD = 5120
B = 1024
NBLK = NPAD // B
THRESH = 0.3


def _nms_body(col_ref, row_ref, keep_ref):
    keep_ref[...] = jnp.ones((1, NPAD), jnp.float32)
    tri = (jax.lax.broadcasted_iota(jnp.int32, (B, B), 1)
           > jax.lax.broadcasted_iota(jnp.int32, (B, B), 0))

    def block(b, carry):
        off = b * B
        rx1 = col_ref[pl.ds(off, B), 0:1]
        ry1 = col_ref[pl.ds(off, B), 1:2]
        rx2 = col_ref[pl.ds(off, B), 2:3]
        ry2 = col_ref[pl.ds(off, B), 3:4]
        rarea = col_ref[pl.ds(off, B), 4:5]

        def iou_chunk(coff):
            cx1 = row_ref[0:1, pl.ds(coff, B)]
            cy1 = row_ref[1:2, pl.ds(coff, B)]
            cx2 = row_ref[2:3, pl.ds(coff, B)]
            cy2 = row_ref[3:4, pl.ds(coff, B)]
            carea = row_ref[4:5, pl.ds(coff, B)]
            iw = jnp.maximum(jnp.minimum(rx2, cx2) - jnp.maximum(rx1, cx1),
                             0.0)
            ih = jnp.maximum(jnp.minimum(ry2, cy2) - jnp.maximum(ry1, cy1),
                             0.0)
            inter = iw * ih
            union = rarea + carea - inter
            return inter / jnp.maximum(union, 1e-9)

        S_bb = jnp.where((iou_chunk(off) > THRESH) & tri,
                         1.0, 0.0).astype(jnp.bfloat16)
        k_in = keep_ref[0:1, pl.ds(off, B)]

        def fcond(state):
            t, _, changed = state
            return changed & (t < B)

        def fbody(state):
            t, k, _ = state
            cnt = jax.lax.dot_general(
                k.astype(jnp.bfloat16), S_bb,
                (((1,), (0,)), ((), ())),
                preferred_element_type=jnp.float32)
            k_new = k_in * jnp.where(cnt > 0.0, 0.0, 1.0)
            return t + 1, k_new, jnp.any(k_new != k)

        _, k_fin, _ = jax.lax.while_loop(fcond, fbody,
                                         (0, k_in, jnp.bool_(True)))
        keep_ref[0:1, pl.ds(off, B)] = k_fin
        k_bf = k_fin.astype(jnp.bfloat16)

        def chunk(c, carry2):
            coff = c * B
            Sc = jnp.where(iou_chunk(coff) > THRESH,
                           1.0, 0.0).astype(jnp.bfloat16)
            sup = jax.lax.dot_general(
                k_bf, Sc, (((1,), (0,)), ((), ())),
                preferred_element_type=jnp.float32)
            keep_ref[0:1, pl.ds(coff, B)] = (
                keep_ref[0:1, pl.ds(coff, B)]
                * jnp.where(sup > 0.0, 0.0, 1.0))
            return carry2

        jax.lax.fori_loop(b + 1, NBLK, chunk, 0)
        return carry

    jax.lax.fori_loop(0, NBLK, block, 0)


def _run_nms(colmat, rowmat, interpret=False):
    return pl.pallas_call(
        _nms_body,
        out_shape=jax.ShapeDtypeStruct((1, NPAD), jnp.float32),
        in_specs=[
            pl.BlockSpec(memory_space=pltpu.VMEM),
            pl.BlockSpec(memory_space=pltpu.VMEM),
        ],
        out_specs=pl.BlockSpec(memory_space=pltpu.VMEM),
        interpret=interpret,
    )(colmat, rowmat)


def kernel(boxes, scores, interpret=False):
    scores = scores.astype(jnp.float32)
    boxes = boxes.astype(jnp.float32)
    scores_p = jnp.concatenate(
        [scores, jnp.full((NPAD - N,), -1.0, jnp.float32)])
    boxes_p = jnp.concatenate(
        [boxes, jnp.zeros((NPAD - N, 4), jnp.float32)], axis=0)
    order = jnp.argsort(-scores_p)
    sb = boxes_p[order]
    x1, y1, x2, y2 = sb[:, 0], sb[:, 1], sb[:, 2], sb[:, 3]
    area = (x2 - x1) * (y2 - y1)
    zero = jnp.zeros((NPAD,), jnp.float32)
    colmat = jnp.stack([x1, y1, x2, y2, area, zero, zero, zero], axis=1)
    rowmat = colmat.T
    keep_sorted = _run_nms(colmat, rowmat, interpret=interpret)[0]
    keep = jnp.zeros((NPAD,), jnp.float32).at[order].set(keep_sorted)
    return scores * keep[:N]
